# Initial kernel scaffold; baseline (speedup 1.0000x reference)
import jax
import jax.numpy as jnp
from jax.experimental import pallas as pl


def kernel(pos, batch, params):
    raise NotImplementedError("write your pallas kernel here")


# speedup vs baseline: 4.5237x; 4.5237x over previous
import functools

import jax
import jax.numpy as jnp
from jax import lax
from jax.experimental import pallas as pl
from jax.experimental.pallas import tpu as pltpu
from jax.experimental.pallas import tpu_sc as plsc

N = 8192
B = 8
K = 20
EPS = 1e-5
BIG = 1e30


_KNN_RB = 256


def _knn_body(xr_ref, xf_ref, sqr_ref, sqc_ref, br_ref, bc_ref, idx_ref):
    xr = xr_ref[...]
    xf = xf_ref[...]
    g = lax.dot_general(xr, xf, (((1,), (1,)), ((), ())),
                        preferred_element_type=jnp.float32)
    d = (sqr_ref[...] - 2.0 * g) + sqc_ref[...]
    d = jnp.where(br_ref[...] != bc_ref[...], BIG, d)

    cols = lax.broadcasted_iota(jnp.int32, (_KNN_RB, N), 1)
    sels = []
    for _ in range(K):
        mn = jnp.min(d, axis=1, keepdims=True)
        sel = jnp.min(jnp.where(d == mn, cols, N), axis=1, keepdims=True)
        sels.append(sel)
        d = jnp.where(cols == sel, jnp.float32(jnp.inf), d)
    idx_ref[...] = jnp.concatenate(sels, axis=1)


def _knn(x, sq, batch_f):
    n, c = x.shape
    grid = n // _KNN_RB
    return pl.pallas_call(
        _knn_body,
        grid=(grid,),
        in_specs=[
            pl.BlockSpec((_KNN_RB, c), lambda i: (i, 0)),
            pl.BlockSpec((n, c), lambda i: (0, 0)),
            pl.BlockSpec((_KNN_RB, 1), lambda i: (i, 0)),
            pl.BlockSpec((1, n), lambda i: (0, 0)),
            pl.BlockSpec((_KNN_RB, 1), lambda i: (i, 0)),
            pl.BlockSpec((1, n), lambda i: (0, 0)),
        ],
        out_specs=pl.BlockSpec((_KNN_RB, K), lambda i: (i, 0)),
        out_shape=jax.ShapeDtypeStruct((n, K), jnp.int32),
    )(x, x, sq.reshape(n, 1), sq.reshape(1, n), batch_f,
      batch_f.reshape(1, n))


def _sc_gather(table, idx2d):
    t_rows, d = table.shape
    n_idx_rows, ipr = idx2d.shape
    m = n_idx_rows * ipr
    info = plsc.get_sparse_core_info()
    nc, ns = info.num_cores, info.num_subcores
    nw = nc * ns
    rows_w = n_idx_rows // nw
    group = max(1, min(rows_w, (160 * 1024) // (ipr * d * 4)))
    while rows_w % group:
        group -= 1
    n_groups = rows_w // group
    mesh = plsc.VectorSubcoreMesh(core_axis_name="c", subcore_axis_name="s")

    @functools.partial(
        pl.kernel, mesh=mesh,
        out_type=jax.ShapeDtypeStruct((m, d), jnp.float32),
        scratch_types=[
            pltpu.VMEM((rows_w, ipr), jnp.int32),
            pltpu.VMEM((group * ipr, d), jnp.float32),
            pltpu.SemaphoreType.DMA,
        ],
    )
    def k(tab_hbm, idx_hbm, out_hbm, idx_v, stage_v, sem):
        wid = lax.axis_index("s") * nc + lax.axis_index("c")
        rbase = wid * rows_w
        pltpu.sync_copy(idx_hbm.at[pl.ds(rbase, rows_w)], idx_v)
        for g in range(n_groups):
            descs = []
            for j in range(group):
                descs.append(pltpu.async_copy(
                    tab_hbm.at[idx_v.at[g * group + j]],
                    stage_v.at[pl.ds(j * ipr, ipr)], sem))
            for de in descs:
                de.wait()
            pltpu.sync_copy(
                stage_v,
                out_hbm.at[pl.ds((rbase + g * group) * ipr, group * ipr)])

    return k(table, idx2d)


_L1_RB = 512


def _seg_body(y_ref, bf_ref, seg_ref):
    @pl.when(pl.program_id(0) == 0)
    def _():
        seg_ref[...] = jnp.full_like(seg_ref, -jnp.inf)

    y = y_ref[...]
    bf = bf_ref[...]
    segs = [jnp.max(jnp.where(bf == float(bb), y, -jnp.inf), axis=0)
            for bb in range(B)]
    seg_ref[...] = jnp.maximum(seg_ref[...], jnp.stack(segs))


def _segmax(y, batch_f):
    h = y.shape[1]
    grid = N // _L1_RB
    return pl.pallas_call(
        _seg_body,
        grid=(grid,),
        in_specs=[
            pl.BlockSpec((_L1_RB, h), lambda i: (i, 0)),
            pl.BlockSpec((_L1_RB, 1), lambda i: (i, 0)),
        ],
        out_specs=pl.BlockSpec((B, h), lambda i: (0, 0)),
        out_shape=jax.ShapeDtypeStruct((B, h), jnp.float32),
    )(y, batch_f)


def _head_body(x_ref, m5_ref, s5_ref, w1_ref, b1_ref, w2_ref, b2_ref,
               w3_ref, b3_ref, out_ref):
    def bn(y):
        m = jnp.mean(y, axis=0, keepdims=True)
        v = jnp.mean((y - m) * (y - m), axis=0, keepdims=True)
        return (y - m) * lax.rsqrt(v + EPS)

    pooled = (x_ref[...] - m5_ref[...]) * s5_ref[...]
    h = bn(jnp.maximum(jnp.dot(pooled, w1_ref[...],
                               preferred_element_type=jnp.float32)
                       + b1_ref[...], 0.0))
    h = bn(jnp.maximum(jnp.dot(h, w2_ref[...],
                               preferred_element_type=jnp.float32)
                       + b2_ref[...], 0.0))
    logits = jnp.dot(h, w3_ref[...],
                     preferred_element_type=jnp.float32) + b3_ref[...]
    zmax = jnp.max(logits, axis=1, keepdims=True)
    sh = logits - zmax
    out_ref[...] = sh - jnp.log(jnp.sum(jnp.exp(sh), axis=1, keepdims=True))


def _head(x, m5, s5, w1, b1, w2, b2, w3, b3):
    return pl.pallas_call(
        _head_body,
        out_shape=jax.ShapeDtypeStruct((B, w3.shape[1]), jnp.float32),
    )(x, m5.reshape(1, -1), s5.reshape(1, -1), w1, b1.reshape(1, -1),
      w2, b2.reshape(1, -1), w3, b3.reshape(1, -1))


def _mlp_bn(y, b, g, be):
    y = jax.nn.relu(y + b)
    m = jnp.mean(y, axis=0)
    v = jnp.var(y, axis=0)
    return g * (y - m) * lax.rsqrt(v + EPS) + be


def kernel(pos, batch, params):
    batch_f = batch.astype(jnp.float32).reshape(N, 1)
    (w1, b1, g1, be1), (w2, b2, g2, be2), (w3, b3, g3, be3) = params["c1"]
    (wc2, bc2, gc2, bec2) = params["c2"][0]
    (wl1, bl1, gl1, bel1) = params["l1"][0]

    sq1 = jnp.sum(pos * pos, axis=1)
    idx1 = _knn(pos, sq1, batch_f)
    pos_pad = jnp.pad(pos, ((0, 0), (0, 125)))
    xj1 = _sc_gather(pos_pad, idx1.reshape(N * K // 128, 128))[:, :3]
    xi1 = jnp.broadcast_to(pos[:, None, :], (N, K, 3)).reshape(N * K, 3)
    feat1 = jnp.concatenate([xi1, xj1 - xi1], axis=1)
    h = _mlp_bn(jnp.dot(feat1, w1), b1, g1, be1)
    h = _mlp_bn(jnp.dot(h, w2), b2, g2, be2)
    h = _mlp_bn(jnp.dot(h, w3), b3, g3, be3)
    x1 = jnp.max(h.reshape(N, K, -1), axis=1)

    sq2 = jnp.sum(x1 * x1, axis=1)
    idx2 = _knn(x1, sq2, batch_f)
    x1_pad = jnp.pad(x1, ((0, 0), (0, 64)))
    xj2 = _sc_gather(x1_pad, idx2.reshape(N * K // 128, 128))[:, :64]
    xi2 = jnp.broadcast_to(x1[:, None, :], (N, K, 64)).reshape(N * K, 64)
    feat2 = jnp.concatenate([xi2, xj2 - xi2], axis=1)
    h = _mlp_bn(jnp.dot(feat2, wc2), bc2, gc2, bec2)
    x2 = jnp.max(h.reshape(N, K, -1), axis=1)

    out = _mlp_bn(jnp.dot(jnp.concatenate([x1, x2], axis=1), wl1),
                  bl1, gl1, bel1)
    pooled = _segmax(out, batch_f)

    return _head(pooled, jnp.zeros((1024,), jnp.float32),
                 jnp.ones((1024,), jnp.float32),
                 *params["h1"][0][:2], *params["h2"][0][:2],
                 params["h3_W"], params["h3_b"])


# speedup vs baseline: 4.9321x; 1.0903x over previous
import functools

import jax
import jax.numpy as jnp
from jax import lax
from jax.experimental import pallas as pl
from jax.experimental.pallas import tpu as pltpu
from jax.experimental.pallas import tpu_sc as plsc

N = 8192
B = 8
K = 20
EPS = 1e-5
BIG = 1e30


_KNN_RB = 512


def _knn_body(xr_ref, xf_ref, sqr_ref, sqc_ref, br_ref, bc_ref, idx_ref):
    xr = xr_ref[...]
    xf = xf_ref[...]
    g = lax.dot_general(xr, xf, (((1,), (1,)), ((), ())),
                        preferred_element_type=jnp.float32)
    d = (sqr_ref[...] - 2.0 * g) + sqc_ref[...]
    d = jnp.where(br_ref[...] != bc_ref[...], BIG, d)

    cols = lax.broadcasted_iota(jnp.int32, (_KNN_RB, N), 1)
    sels = []
    for _ in range(K):
        mn = jnp.min(d, axis=1, keepdims=True)
        sel = jnp.min(jnp.where(d == mn, cols, N), axis=1, keepdims=True)
        sels.append(sel)
        d = jnp.where(cols == sel, jnp.float32(jnp.inf), d)
    idx_ref[...] = jnp.concatenate(sels, axis=1)


def _knn(x, sq, batch_f):
    n, c = x.shape
    grid = n // _KNN_RB
    return pl.pallas_call(
        _knn_body,
        grid=(grid,),
        in_specs=[
            pl.BlockSpec((_KNN_RB, c), lambda i: (i, 0)),
            pl.BlockSpec((n, c), lambda i: (0, 0)),
            pl.BlockSpec((_KNN_RB, 1), lambda i: (i, 0)),
            pl.BlockSpec((1, n), lambda i: (0, 0)),
            pl.BlockSpec((_KNN_RB, 1), lambda i: (i, 0)),
            pl.BlockSpec((1, n), lambda i: (0, 0)),
        ],
        out_specs=pl.BlockSpec((_KNN_RB, K), lambda i: (i, 0)),
        out_shape=jax.ShapeDtypeStruct((n, K), jnp.int32),
    )(x, x, sq.reshape(n, 1), sq.reshape(1, n), batch_f,
      batch_f.reshape(1, n))


def _sc_gather(table, idx2d):
    t_rows, d = table.shape
    n_idx_rows, ipr = idx2d.shape
    m = n_idx_rows * ipr
    info = plsc.get_sparse_core_info()
    nc, ns = info.num_cores, info.num_subcores
    nw = nc * ns
    rows_w = n_idx_rows // nw
    group = max(1, min(rows_w, (160 * 1024) // (ipr * d * 4)))
    while rows_w % group:
        group -= 1
    n_groups = rows_w // group
    mesh = plsc.VectorSubcoreMesh(core_axis_name="c", subcore_axis_name="s")

    @functools.partial(
        pl.kernel, mesh=mesh,
        out_type=jax.ShapeDtypeStruct((m, d), jnp.float32),
        scratch_types=[
            pltpu.VMEM((rows_w, ipr), jnp.int32),
            pltpu.VMEM((group * ipr, d), jnp.float32),
            pltpu.SemaphoreType.DMA,
        ],
    )
    def k(tab_hbm, idx_hbm, out_hbm, idx_v, stage_v, sem):
        wid = lax.axis_index("s") * nc + lax.axis_index("c")
        rbase = wid * rows_w
        pltpu.sync_copy(idx_hbm.at[pl.ds(rbase, rows_w)], idx_v)
        for g in range(n_groups):
            descs = []
            for j in range(group):
                descs.append(pltpu.async_copy(
                    tab_hbm.at[idx_v.at[g * group + j]],
                    stage_v.at[pl.ds(j * ipr, ipr)], sem))
            for de in descs:
                de.wait()
            pltpu.sync_copy(
                stage_v,
                out_hbm.at[pl.ds((rbase + g * group) * ipr, group * ipr)])

    return k(table, idx2d)


_L1_RB = 512


def _seg_body(y_ref, bf_ref, seg_ref):
    @pl.when(pl.program_id(0) == 0)
    def _():
        seg_ref[...] = jnp.full_like(seg_ref, -jnp.inf)

    y = y_ref[...]
    bf = bf_ref[...]
    segs = [jnp.max(jnp.where(bf == float(bb), y, -jnp.inf), axis=0)
            for bb in range(B)]
    seg_ref[...] = jnp.maximum(seg_ref[...], jnp.stack(segs))


def _segmax(y, batch_f):
    h = y.shape[1]
    grid = N // _L1_RB
    return pl.pallas_call(
        _seg_body,
        grid=(grid,),
        in_specs=[
            pl.BlockSpec((_L1_RB, h), lambda i: (i, 0)),
            pl.BlockSpec((_L1_RB, 1), lambda i: (i, 0)),
        ],
        out_specs=pl.BlockSpec((B, h), lambda i: (0, 0)),
        out_shape=jax.ShapeDtypeStruct((B, h), jnp.float32),
    )(y, batch_f)


def _head_body(x_ref, m5_ref, s5_ref, w1_ref, b1_ref, w2_ref, b2_ref,
               w3_ref, b3_ref, out_ref):
    def bn(y):
        m = jnp.mean(y, axis=0, keepdims=True)
        v = jnp.mean((y - m) * (y - m), axis=0, keepdims=True)
        return (y - m) * lax.rsqrt(v + EPS)

    pooled = (x_ref[...] - m5_ref[...]) * s5_ref[...]
    h = bn(jnp.maximum(jnp.dot(pooled, w1_ref[...],
                               preferred_element_type=jnp.float32)
                       + b1_ref[...], 0.0))
    h = bn(jnp.maximum(jnp.dot(h, w2_ref[...],
                               preferred_element_type=jnp.float32)
                       + b2_ref[...], 0.0))
    logits = jnp.dot(h, w3_ref[...],
                     preferred_element_type=jnp.float32) + b3_ref[...]
    zmax = jnp.max(logits, axis=1, keepdims=True)
    sh = logits - zmax
    out_ref[...] = sh - jnp.log(jnp.sum(jnp.exp(sh), axis=1, keepdims=True))


def _head(x, m5, s5, w1, b1, w2, b2, w3, b3):
    return pl.pallas_call(
        _head_body,
        out_shape=jax.ShapeDtypeStruct((B, w3.shape[1]), jnp.float32),
    )(x, m5.reshape(1, -1), s5.reshape(1, -1), w1, b1.reshape(1, -1),
      w2, b2.reshape(1, -1), w3, b3.reshape(1, -1))


def _mlp_bn(y, b, g, be):
    y = jax.nn.relu(y + b)
    m = jnp.mean(y, axis=0)
    v = jnp.var(y, axis=0)
    return g * (y - m) * lax.rsqrt(v + EPS) + be


def kernel(pos, batch, params):
    batch_f = batch.astype(jnp.float32).reshape(N, 1)
    (w1, b1, g1, be1), (w2, b2, g2, be2), (w3, b3, g3, be3) = params["c1"]
    (wc2, bc2, gc2, bec2) = params["c2"][0]
    (wl1, bl1, gl1, bel1) = params["l1"][0]

    sq1 = jnp.sum(pos * pos, axis=1)
    idx1 = _knn(pos, sq1, batch_f)
    pos_pad = jnp.pad(pos, ((0, 0), (0, 125)))
    xj1 = _sc_gather(pos_pad, idx1.reshape(N * K // 128, 128))[:, :3]
    xi1 = jnp.broadcast_to(pos[:, None, :], (N, K, 3)).reshape(N * K, 3)
    feat1 = jnp.concatenate([xi1, xj1 - xi1], axis=1)
    h = _mlp_bn(jnp.dot(feat1, w1), b1, g1, be1)
    h = _mlp_bn(jnp.dot(h, w2), b2, g2, be2)
    h = _mlp_bn(jnp.dot(h, w3), b3, g3, be3)
    x1 = jnp.max(h.reshape(N, K, -1), axis=1)

    sq2 = jnp.sum(x1 * x1, axis=1)
    idx2 = _knn(x1, sq2, batch_f)
    x1_pad = jnp.pad(x1, ((0, 0), (0, 64)))
    xj2 = _sc_gather(x1_pad, idx2.reshape(N * K // 128, 128))[:, :64]
    xi2 = jnp.broadcast_to(x1[:, None, :], (N, K, 64)).reshape(N * K, 64)
    feat2 = jnp.concatenate([xi2, xj2 - xi2], axis=1)
    h = _mlp_bn(jnp.dot(feat2, wc2), bc2, gc2, bec2)
    x2 = jnp.max(h.reshape(N, K, -1), axis=1)

    out = _mlp_bn(jnp.dot(jnp.concatenate([x1, x2], axis=1), wl1),
                  bl1, gl1, bel1)
    pooled = _segmax(out, batch_f)

    return _head(pooled, jnp.zeros((1024,), jnp.float32),
                 jnp.ones((1024,), jnp.float32),
                 *params["h1"][0][:2], *params["h2"][0][:2],
                 params["h3_W"], params["h3_b"])
